# baseline (speedup 1.0000x reference)
import functools

import jax
import jax.numpy as jnp
from jax import lax
from jax.experimental import pallas as pl
from jax.experimental.pallas import tpu as pltpu
from jax.experimental.pallas import tpu_sc as plsc

_SCORE_THRESH = 0.25
_NMS_THRESH = 0.45
_DETECTIONS = 300
_N = 5000
_L = 16
_R = 64
_C = 80
_NTOT = _R * _C
_NCHUNK = _NTOT // _L
_KPAD = 304
_NV = _R // _L
_IMAX = 2147483647

_mesh = plsc.VectorSubcoreMesh(core_axis_name="c", subcore_axis_name="s")


def _iou_vs_chunk(bx1, by1, bx2, by2, barea, x1, y1, x2, y2, area):
    ltx = jnp.maximum(bx1, x1)
    lty = jnp.maximum(by1, y1)
    rbx = jnp.minimum(bx2, x2)
    rby = jnp.minimum(by2, y2)
    w = jnp.maximum(rbx - ltx, jnp.float32(0.0))
    h = jnp.maximum(rby - lty, jnp.float32(0.0))
    inter = w * h
    return inter / (barea + area - inter + jnp.float32(1e-7))


@functools.partial(
    pl.kernel,
    out_type=jax.ShapeDtypeStruct((8, _KPAD), jnp.float32),
    mesh=_mesh,
    scratch_types=[
        pltpu.VMEM((5, _NTOT), jnp.float32),
        pltpu.VMEM((_NTOT,), jnp.int32),
        pltpu.VMEM((_L,), jnp.float32),
        pltpu.VMEM((8, _KPAD), jnp.float32),
        pltpu.VMEM((8, _L), jnp.int32),
        pltpu.VMEM((_NV, _L), jnp.float32),
        pltpu.SMEM((1,), jnp.int32),
    ],
    compiler_params=pltpu.CompilerParams(needs_layout_passes=False),
)
def _nms_sc(tab_hbm, si_hbm, out_hbm, tab_v, si_v, sup_v, kept_v, mi_v, mf_v,
            nk_s):
    cid = lax.axis_index("c")
    sid = lax.axis_index("s")

    @pl.when((cid == 0) & (sid == 0))
    def _():
        zero16 = jnp.zeros((_L,), jnp.float32)
        for r in range(8):
            def _z(j, _, r=r):
                kept_v[r, pl.ds(j * _L, _L)] = zero16
                return 0
            lax.fori_loop(0, _KPAD // _L, _z, 0)

        rid = lax.iota(jnp.int32, _L)
        rowc = [jnp.full((_L,), r, jnp.int32) for r in range(6)]
        nk_s[0] = jnp.int32(0)
        pltpu.sync_copy(tab_hbm, tab_v)
        pltpu.sync_copy(si_hbm, si_v)

        sbase = [rid * _C + jnp.int32(v * _L * _C) for v in range(_NV)]
        for v in range(_NV):
            mi_v[v, :] = jnp.zeros((_L,), jnp.int32)
            hi = plsc.load_gather(si_v, [sbase[v]])
            mi_v[_NV + v, :] = hi
            mf_v[v, :] = plsc.load_gather(tab_v, [rowc[4], hi])

        def chunk(c, carry):
            @pl.when(nk_s[0] < _DETECTIONS)
            def _():
                nk0 = nk_s[0]
                ptr = [mi_v[v, :] for v in range(_NV)]
                hidx = [mi_v[_NV + v, :] for v in range(_NV)]
                hsc = [mf_v[v, :] for v in range(_NV)]

                ovec = jnp.zeros((_L,), jnp.int32)
                svec = jnp.full((_L,), -2.0, jnp.float32)
                for t in range(_L):
                    gm = jnp.max(jnp.maximum(jnp.maximum(hsc[0], hsc[1]),
                                             jnp.maximum(hsc[2], hsc[3])))
                    gmb = jnp.full((_L,), gm, jnp.float32)
                    cand = [jnp.where(hsc[v] == gmb, hidx[v], jnp.int32(_IMAX))
                            for v in range(_NV)]
                    imin = jnp.min(jnp.minimum(jnp.minimum(cand[0], cand[1]),
                                               jnp.minimum(cand[2], cand[3])))
                    iminb = jnp.full((_L,), imin, jnp.int32)
                    ovec = jnp.where(rid == t, iminb, ovec)
                    svec = jnp.where(rid == t, gmb, svec)
                    for v in range(_NV):
                        pop = cand[v] == iminb
                        ptr[v] = ptr[v] + pop.astype(jnp.int32)
                        pos = sbase[v] + jnp.minimum(ptr[v], jnp.int32(_C - 1))
                        hidx[v] = jnp.where(pop, plsc.load_gather(si_v, [pos]),
                                            hidx[v])
                        hs_new = plsc.load_gather(tab_v, [rowc[4], hidx[v]])
                        hs_new = jnp.where(ptr[v] >= _C,
                                           jnp.float32(-2.0), hs_new)
                        hsc[v] = jnp.where(pop, hs_new, hsc[v])

                for v in range(_NV):
                    mi_v[v, :] = ptr[v]
                    mi_v[_NV + v, :] = hidx[v]
                    mf_v[v, :] = hsc[v]

                x1 = plsc.load_gather(tab_v, [rowc[0], ovec])
                y1 = plsc.load_gather(tab_v, [rowc[1], ovec])
                x2 = plsc.load_gather(tab_v, [rowc[2], ovec])
                y2 = plsc.load_gather(tab_v, [rowc[3], ovec])
                s = svec
                area = (x2 - x1) * (y2 - y1)

                sup = jnp.where(s <= jnp.float32(_SCORE_THRESH),
                                jnp.float32(1.0), jnp.float32(0.0))

                def kbody(k, sup):
                    kv = jnp.full((_L,), k, jnp.int32)
                    kx1 = plsc.load_gather(kept_v, [rowc[0], kv])
                    ky1 = plsc.load_gather(kept_v, [rowc[1], kv])
                    kx2 = plsc.load_gather(kept_v, [rowc[2], kv])
                    ky2 = plsc.load_gather(kept_v, [rowc[3], kv])
                    ka = plsc.load_gather(kept_v, [rowc[5], kv])
                    iou = _iou_vs_chunk(kx1, ky1, kx2, ky2, ka,
                                        x1, y1, x2, y2, area)
                    return jnp.where(iou > jnp.float32(_NMS_THRESH),
                                     jnp.float32(1.0), sup)

                sup = lax.fori_loop(0, nk0, kbody, sup)
                sup_v[...] = sup

                nk = nk0
                for i in range(_L):
                    supc = sup_v[...]
                    keep = (supc[i] == jnp.float32(0.0)) & (nk < _DETECTIONS)

                    @pl.when(keep)
                    def _(i=i, nk=nk):
                        bx1 = x1[i]
                        by1 = y1[i]
                        bx2 = x2[i]
                        by2 = y2[i]
                        ba = area[i]
                        lane = rid == i
                        nkv = jnp.full((_L,), nk, jnp.int32)
                        plsc.store_scatter(kept_v, [rowc[0], nkv], x1, mask=lane)
                        plsc.store_scatter(kept_v, [rowc[1], nkv], y1, mask=lane)
                        plsc.store_scatter(kept_v, [rowc[2], nkv], x2, mask=lane)
                        plsc.store_scatter(kept_v, [rowc[3], nkv], y2, mask=lane)
                        plsc.store_scatter(kept_v, [rowc[4], nkv], s, mask=lane)
                        plsc.store_scatter(kept_v, [rowc[5], nkv], area, mask=lane)
                        iou = _iou_vs_chunk(bx1, by1, bx2, by2, ba,
                                            x1, y1, x2, y2, area)
                        sup_v[...] = jnp.where(iou > jnp.float32(_NMS_THRESH),
                                               jnp.float32(1.0), sup_v[...])

                    nk = jnp.where(keep, nk + jnp.int32(1), nk)

                nk_s[0] = nk

                @pl.when(svec[0] <= jnp.float32(_SCORE_THRESH))
                def _():
                    nk_s[0] = jnp.int32(_DETECTIONS)

            return carry

        lax.fori_loop(0, _NCHUNK, chunk, jnp.int32(0))
        pltpu.sync_copy(kept_v, out_hbm)


def kernel(boxes, scores):
    sp = jnp.pad(scores, (0, _NTOT - _N), constant_values=-1.0)
    iv = jnp.arange(_NTOT, dtype=jnp.int32).reshape(_R, _C)
    _, si = lax.sort_key_val(-sp.reshape(_R, _C), iv)
    b5 = jnp.concatenate([boxes, scores[:, None]], axis=1)
    tab = jnp.concatenate([jnp.pad(b5[:, :4], ((0, _NTOT - _N), (0, 0))),
                           sp[:, None]], axis=1).T
    out = _nms_sc(tab, si.reshape(-1))
    return out[:5, :_DETECTIONS].T


# speedup vs baseline: 1.0841x; 1.0841x over previous
import functools

import jax
import jax.numpy as jnp
from jax import lax
from jax.experimental import pallas as pl
from jax.experimental.pallas import tpu as pltpu
from jax.experimental.pallas import tpu_sc as plsc

_SCORE_THRESH = 0.25
_NMS_THRESH = 0.45
_DETECTIONS = 300
_N = 5000
_L = 16
_R = 64
_C = 80
_CP = 112
_NTOT = _R * _C
_NSTG = _R * _CP
_NCHUNK = _NTOT // _L
_KPAD = 304
_NV = _R // _L

_mesh = plsc.VectorSubcoreMesh(core_axis_name="c", subcore_axis_name="s")


def _perm(x, idx):
    return x.at[idx].get(mode="promise_in_bounds")


def _iou_vs_chunk(bx1, by1, bx2, by2, barea, x1, y1, x2, y2, area):
    ltx = jnp.maximum(bx1, x1)
    lty = jnp.maximum(by1, y1)
    rbx = jnp.minimum(bx2, x2)
    rby = jnp.minimum(by2, y2)
    w = jnp.maximum(rbx - ltx, jnp.float32(0.0))
    h = jnp.maximum(rby - lty, jnp.float32(0.0))
    inter = w * h
    return inter / (barea + area - inter + jnp.float32(1e-7))


@functools.partial(
    pl.kernel,
    out_type=jax.ShapeDtypeStruct((8, _KPAD), jnp.float32),
    mesh=_mesh,
    scratch_types=[
        pltpu.VMEM((5, _NTOT), jnp.float32),
        pltpu.VMEM((_NSTG,), jnp.int32),
        pltpu.VMEM((_NSTG,), jnp.float32),
        pltpu.VMEM((_L,), jnp.float32),
        pltpu.VMEM((8, _KPAD), jnp.float32),
        pltpu.VMEM((_NV, _L), jnp.int32),
        pltpu.VMEM((_NV, _L), jnp.float32),
        pltpu.SMEM((1,), jnp.int32),
    ],
    compiler_params=pltpu.CompilerParams(needs_layout_passes=False),
)
def _nms_sc(tab_hbm, si_hbm, ss_hbm, out_hbm, tab_v, si_v, ss_v, sup_v, kept_v,
            mp_v, mh_v, nk_s):
    cid = lax.axis_index("c")
    sid = lax.axis_index("s")

    @pl.when((cid == 0) & (sid == 0))
    def _():
        zero16 = jnp.zeros((_L,), jnp.float32)
        for r in range(8):
            def _z(j, _, r=r):
                kept_v[r, pl.ds(j * _L, _L)] = zero16
                return 0
            lax.fori_loop(0, _KPAD // _L, _z, 0)

        rid = lax.iota(jnp.int32, _L)
        rowc = [jnp.full((_L,), r, jnp.int32) for r in range(6)]
        bfly = [rid ^ d for d in (1, 2, 4, 8)]
        nk_s[0] = jnp.int32(0)
        pltpu.sync_copy(tab_hbm, tab_v)
        pltpu.sync_copy(si_hbm, si_v)
        pltpu.sync_copy(ss_hbm, ss_v)

        sbase = [rid * _CP + jnp.int32(v * _L * _CP) for v in range(_NV)]
        for v in range(_NV):
            mp_v[v, :] = sbase[v]
            mh_v[v, :] = plsc.load_gather(ss_v, [sbase[v]])

        def chunk(c, carry):
            @pl.when(nk_s[0] < _DETECTIONS)
            def _():
                nk0 = nk_s[0]
                posv = [mp_v[v, :] for v in range(_NV)]
                hsc = [mh_v[v, :] for v in range(_NV)]

                povec = jnp.zeros((_L,), jnp.int32)
                svec = jnp.full((_L,), -2.0, jnp.float32)
                for t in range(_L):
                    gm = jnp.maximum(jnp.maximum(hsc[0], hsc[1]),
                                     jnp.maximum(hsc[2], hsc[3]))
                    for bf in bfly:
                        gm = jnp.maximum(gm, _perm(gm, bf))
                    m = [hsc[v] == gm for v in range(_NV)]
                    cnt = [plsc.all_reduce_population_count(m[v])
                           for v in range(_NV)]
                    ffs = [plsc.all_reduce_ffs(m[v]) for v in range(_NV)]
                    sv = jnp.where(
                        cnt[0] > 0, ffs[0],
                        jnp.where(cnt[1] > 0, ffs[1] + _L,
                                  jnp.where(cnt[2] > 0, ffs[2] + 2 * _L,
                                            ffs[3] + 3 * _L)))
                    psel = jnp.where(
                        sv < _L, posv[0],
                        jnp.where(sv < 2 * _L, posv[1],
                                  jnp.where(sv < 3 * _L, posv[2], posv[3])))
                    ppos = _perm(psel, sv & (_L - 1))
                    povec = jnp.where(rid == t, ppos, povec)
                    svec = jnp.where(rid == t, gm, svec)
                    for v in range(_NV):
                        pop = (rid + v * _L) == sv
                        posv[v] = posv[v] + pop.astype(jnp.int32)
                        hs_new = plsc.load_gather(ss_v, [posv[v]])
                        hsc[v] = jnp.where(pop, hs_new, hsc[v])

                for v in range(_NV):
                    mp_v[v, :] = posv[v]
                    mh_v[v, :] = hsc[v]

                ovec = plsc.load_gather(si_v, [povec])
                x1 = plsc.load_gather(tab_v, [rowc[0], ovec])
                y1 = plsc.load_gather(tab_v, [rowc[1], ovec])
                x2 = plsc.load_gather(tab_v, [rowc[2], ovec])
                y2 = plsc.load_gather(tab_v, [rowc[3], ovec])
                s = svec
                area = (x2 - x1) * (y2 - y1)

                sup = jnp.where(s <= jnp.float32(_SCORE_THRESH),
                                jnp.float32(1.0), jnp.float32(0.0))

                def kbody(k, sup):
                    kv = jnp.full((_L,), k, jnp.int32)
                    kx1 = plsc.load_gather(kept_v, [rowc[0], kv])
                    ky1 = plsc.load_gather(kept_v, [rowc[1], kv])
                    kx2 = plsc.load_gather(kept_v, [rowc[2], kv])
                    ky2 = plsc.load_gather(kept_v, [rowc[3], kv])
                    ka = plsc.load_gather(kept_v, [rowc[5], kv])
                    iou = _iou_vs_chunk(kx1, ky1, kx2, ky2, ka,
                                        x1, y1, x2, y2, area)
                    return jnp.where(iou > jnp.float32(_NMS_THRESH),
                                     jnp.float32(1.0), sup)

                sup = lax.fori_loop(0, nk0, kbody, sup)
                sup_v[...] = sup

                nk = nk0
                for i in range(_L):
                    supc = sup_v[...]
                    keep = (supc[i] == jnp.float32(0.0)) & (nk < _DETECTIONS)

                    @pl.when(keep)
                    def _(i=i, nk=nk):
                        bx1 = x1[i]
                        by1 = y1[i]
                        bx2 = x2[i]
                        by2 = y2[i]
                        ba = area[i]
                        lane = rid == i
                        nkv = jnp.full((_L,), nk, jnp.int32)
                        plsc.store_scatter(kept_v, [rowc[0], nkv], x1, mask=lane)
                        plsc.store_scatter(kept_v, [rowc[1], nkv], y1, mask=lane)
                        plsc.store_scatter(kept_v, [rowc[2], nkv], x2, mask=lane)
                        plsc.store_scatter(kept_v, [rowc[3], nkv], y2, mask=lane)
                        plsc.store_scatter(kept_v, [rowc[4], nkv], s, mask=lane)
                        plsc.store_scatter(kept_v, [rowc[5], nkv], area, mask=lane)
                        iou = _iou_vs_chunk(bx1, by1, bx2, by2, ba,
                                            x1, y1, x2, y2, area)
                        sup_v[...] = jnp.where(iou > jnp.float32(_NMS_THRESH),
                                               jnp.float32(1.0), sup_v[...])

                    nk = jnp.where(keep, nk + jnp.int32(1), nk)

                nk_s[0] = nk

                @pl.when(svec[0] <= jnp.float32(_SCORE_THRESH))
                def _():
                    nk_s[0] = jnp.int32(_DETECTIONS)

            return carry

        lax.fori_loop(0, _NCHUNK, chunk, jnp.int32(0))
        pltpu.sync_copy(kept_v, out_hbm)


def kernel(boxes, scores):
    sp = jnp.pad(scores, (0, _NTOT - _N), constant_values=-1.0)
    iv = jnp.arange(_NTOT, dtype=jnp.int32).reshape(_R, _C)
    sk, si = lax.sort_key_val(-sp.reshape(_R, _C), iv)
    pad_s = jnp.full((_R, _CP - _C), -2.0, jnp.float32)
    pad_i = jnp.full((_R, _CP - _C), _NTOT - 1, jnp.int32)
    ss = jnp.concatenate([-sk, pad_s], axis=1).reshape(-1)
    sip = jnp.concatenate([si, pad_i], axis=1).reshape(-1)
    b5 = jnp.concatenate([boxes, scores[:, None]], axis=1)
    tab = jnp.concatenate([jnp.pad(b5[:, :4], ((0, _NTOT - _N), (0, 0))),
                           sp[:, None]], axis=1).T
    out = _nms_sc(tab, sip, ss)
    return out[:5, :_DETECTIONS].T


# speedup vs baseline: 1.0950x; 1.0100x over previous
import functools

import jax
import jax.numpy as jnp
from jax import lax
from jax.experimental import pallas as pl
from jax.experimental.pallas import tpu as pltpu
from jax.experimental.pallas import tpu_sc as plsc

_SCORE_THRESH = 0.25
_NMS_THRESH = 0.45
_DETECTIONS = 300
_N = 5000
_L = 16
_R = 64
_C = 80
_CP = 112
_NTOT = _R * _C
_NSTG = _R * _CP
_NCHUNK = _NTOT // _L
_KPAD = 304
_NV = _R // _L

_mesh = plsc.VectorSubcoreMesh(core_axis_name="c", subcore_axis_name="s")


def _perm(x, idx):
    return x.at[idx].get(mode="promise_in_bounds")


def _iou_vs_chunk(bx1, by1, bx2, by2, barea, x1, y1, x2, y2, area):
    ltx = jnp.maximum(bx1, x1)
    lty = jnp.maximum(by1, y1)
    rbx = jnp.minimum(bx2, x2)
    rby = jnp.minimum(by2, y2)
    w = jnp.maximum(rbx - ltx, jnp.float32(0.0))
    h = jnp.maximum(rby - lty, jnp.float32(0.0))
    inter = w * h
    return inter / (barea + area - inter + jnp.float32(1e-7))


@functools.partial(
    pl.kernel,
    out_type=jax.ShapeDtypeStruct((8, _KPAD), jnp.float32),
    mesh=_mesh,
    scratch_types=[
        pltpu.VMEM((5, _NTOT), jnp.float32),
        pltpu.VMEM((_NSTG,), jnp.int32),
        pltpu.VMEM((_NSTG,), jnp.float32),
        pltpu.VMEM((_L,), jnp.float32),
        pltpu.VMEM((8, _KPAD), jnp.float32),
        pltpu.VMEM((_NV, _L), jnp.int32),
        pltpu.VMEM((_NV, _L), jnp.float32),
        pltpu.SMEM((1,), jnp.int32),
    ],
    compiler_params=pltpu.CompilerParams(needs_layout_passes=False),
)
def _nms_sc(tab_hbm, si_hbm, ss_hbm, out_hbm, tab_v, si_v, ss_v, sup_v, kept_v,
            mp_v, mh_v, nk_s):
    cid = lax.axis_index("c")
    sid = lax.axis_index("s")

    @pl.when((cid == 0) & (sid == 0))
    def _():
        zero16 = jnp.zeros((_L,), jnp.float32)
        for r in range(8):
            def _z(j, _, r=r):
                kept_v[r, pl.ds(j * _L, _L)] = zero16
                return 0
            lax.fori_loop(0, _KPAD // _L, _z, 0)

        rid = lax.iota(jnp.int32, _L)
        rowc = [jnp.full((_L,), r, jnp.int32) for r in range(6)]
        bfly = [rid ^ d for d in (1, 2, 4, 8)]
        nk_s[0] = jnp.int32(0)
        pltpu.sync_copy(tab_hbm, tab_v)
        pltpu.sync_copy(si_hbm, si_v)
        pltpu.sync_copy(ss_hbm, ss_v)

        sbase = [rid * _CP + jnp.int32(v * _L * _CP) for v in range(_NV)]
        for v in range(_NV):
            mp_v[v, :] = sbase[v]
            mh_v[v, :] = plsc.load_gather(ss_v, [sbase[v]])

        def chunk(c, carry):
            @pl.when(nk_s[0] < _DETECTIONS)
            def _():
                nk0 = nk_s[0]
                posv = [mp_v[v, :] for v in range(_NV)]
                hsc = [mh_v[v, :] for v in range(_NV)]

                povec = jnp.zeros((_L,), jnp.int32)
                svec = jnp.full((_L,), -2.0, jnp.float32)
                for t in range(_L):
                    acc = list(hsc)
                    while len(acc) > 1:
                        acc = [jnp.maximum(acc[2 * a], acc[2 * a + 1])
                               for a in range(len(acc) // 2)]
                    gm = acc[0]
                    for bf in bfly:
                        gm = jnp.maximum(gm, _perm(gm, bf))
                    m = [hsc[v] == gm for v in range(_NV)]
                    cnt = [plsc.all_reduce_population_count(m[v])
                           for v in range(_NV)]
                    ffs = [plsc.all_reduce_ffs(m[v]) for v in range(_NV)]
                    sv = ffs[_NV - 1] + (_NV - 1) * _L
                    for v in reversed(range(_NV - 1)):
                        sv = jnp.where(cnt[v] > 0, ffs[v] + v * _L, sv)
                    psel = posv[_NV - 1]
                    for v in reversed(range(_NV - 1)):
                        psel = jnp.where(sv < (v + 1) * _L, posv[v], psel)
                    ppos = _perm(psel, sv & (_L - 1))
                    povec = jnp.where(rid == t, ppos, povec)
                    svec = jnp.where(rid == t, gm, svec)
                    for v in range(_NV):
                        pop = (rid + v * _L) == sv
                        posv[v] = posv[v] + pop.astype(jnp.int32)
                        hs_new = plsc.load_gather(ss_v, [posv[v]])
                        hsc[v] = jnp.where(pop, hs_new, hsc[v])

                for v in range(_NV):
                    mp_v[v, :] = posv[v]
                    mh_v[v, :] = hsc[v]

                ovec = plsc.load_gather(si_v, [povec])
                x1 = plsc.load_gather(tab_v, [rowc[0], ovec])
                y1 = plsc.load_gather(tab_v, [rowc[1], ovec])
                x2 = plsc.load_gather(tab_v, [rowc[2], ovec])
                y2 = plsc.load_gather(tab_v, [rowc[3], ovec])
                s = svec
                area = (x2 - x1) * (y2 - y1)

                sup = jnp.where(s <= jnp.float32(_SCORE_THRESH),
                                jnp.float32(1.0), jnp.float32(0.0))

                def kbody(k, sup):
                    for u in range(4):
                        kv = jnp.full((_L,), 4 * k + u, jnp.int32)
                        kx1 = plsc.load_gather(kept_v, [rowc[0], kv])
                        ky1 = plsc.load_gather(kept_v, [rowc[1], kv])
                        kx2 = plsc.load_gather(kept_v, [rowc[2], kv])
                        ky2 = plsc.load_gather(kept_v, [rowc[3], kv])
                        ka = plsc.load_gather(kept_v, [rowc[5], kv])
                        iou = _iou_vs_chunk(kx1, ky1, kx2, ky2, ka,
                                            x1, y1, x2, y2, area)
                        sup = jnp.where(iou > jnp.float32(_NMS_THRESH),
                                        jnp.float32(1.0), sup)
                    return sup

                sup = lax.fori_loop(0, (nk0 + 3) // 4, kbody, sup)
                sup_v[...] = sup

                nk = nk0
                for i in range(_L):
                    supc = sup_v[...]
                    keep = (supc[i] == jnp.float32(0.0)) & (nk < _DETECTIONS)

                    @pl.when(keep)
                    def _(i=i, nk=nk):
                        bx1 = x1[i]
                        by1 = y1[i]
                        bx2 = x2[i]
                        by2 = y2[i]
                        ba = area[i]
                        lane = rid == i
                        nkv = jnp.full((_L,), nk, jnp.int32)
                        plsc.store_scatter(kept_v, [rowc[0], nkv], x1, mask=lane)
                        plsc.store_scatter(kept_v, [rowc[1], nkv], y1, mask=lane)
                        plsc.store_scatter(kept_v, [rowc[2], nkv], x2, mask=lane)
                        plsc.store_scatter(kept_v, [rowc[3], nkv], y2, mask=lane)
                        plsc.store_scatter(kept_v, [rowc[4], nkv], s, mask=lane)
                        plsc.store_scatter(kept_v, [rowc[5], nkv], area, mask=lane)
                        iou = _iou_vs_chunk(bx1, by1, bx2, by2, ba,
                                            x1, y1, x2, y2, area)
                        sup_v[...] = jnp.where(iou > jnp.float32(_NMS_THRESH),
                                               jnp.float32(1.0), sup_v[...])

                    nk = jnp.where(keep, nk + jnp.int32(1), nk)

                nk_s[0] = nk

                @pl.when(svec[0] <= jnp.float32(_SCORE_THRESH))
                def _():
                    nk_s[0] = jnp.int32(_DETECTIONS)

            return carry

        lax.fori_loop(0, _NCHUNK, chunk, jnp.int32(0))
        pltpu.sync_copy(kept_v, out_hbm)


def kernel(boxes, scores):
    sp = jnp.pad(scores, (0, _NTOT - _N), constant_values=-1.0)
    iv = jnp.arange(_NTOT, dtype=jnp.int32).reshape(_R, _C)
    sk, si = lax.sort_key_val(-sp.reshape(_R, _C), iv)
    pad_s = jnp.full((_R, _CP - _C), -2.0, jnp.float32)
    pad_i = jnp.full((_R, _CP - _C), _NTOT - 1, jnp.int32)
    ss = jnp.concatenate([-sk, pad_s], axis=1).reshape(-1)
    sip = jnp.concatenate([si, pad_i], axis=1).reshape(-1)
    b5 = jnp.concatenate([boxes, scores[:, None]], axis=1)
    tab = jnp.concatenate([jnp.pad(b5[:, :4], ((0, _NTOT - _N), (0, 0))),
                           sp[:, None]], axis=1).T
    out = _nms_sc(tab, sip, ss)
    return out[:5, :_DETECTIONS].T


# speedup vs baseline: 1.1260x; 1.0284x over previous
import functools

import jax
import jax.numpy as jnp
from jax import lax
from jax.experimental import pallas as pl
from jax.experimental.pallas import tpu as pltpu
from jax.experimental.pallas import tpu_sc as plsc

_SCORE_THRESH = 0.25
_NMS_THRESH = 0.45
_DETECTIONS = 300
_N = 5000
_L = 16
_R = 32
_C = 160
_CP = 192
_NTOT = _R * _C
_NSTG = _R * _CP
_NCHUNK = _NTOT // _L
_KPAD = 304
_NV = _R // _L

_mesh = plsc.VectorSubcoreMesh(core_axis_name="c", subcore_axis_name="s")


def _perm(x, idx):
    return x.at[idx].get(mode="promise_in_bounds")


def _iou_vs_chunk(bx1, by1, bx2, by2, barea, x1, y1, x2, y2, area):
    ltx = jnp.maximum(bx1, x1)
    lty = jnp.maximum(by1, y1)
    rbx = jnp.minimum(bx2, x2)
    rby = jnp.minimum(by2, y2)
    w = jnp.maximum(rbx - ltx, jnp.float32(0.0))
    h = jnp.maximum(rby - lty, jnp.float32(0.0))
    inter = w * h
    return inter / (barea + area - inter + jnp.float32(1e-7))


@functools.partial(
    pl.kernel,
    out_type=jax.ShapeDtypeStruct((8, _KPAD), jnp.float32),
    mesh=_mesh,
    scratch_types=[
        pltpu.VMEM((5, _NTOT), jnp.float32),
        pltpu.VMEM((_NSTG,), jnp.int32),
        pltpu.VMEM((_NSTG,), jnp.float32),
        pltpu.VMEM((_L,), jnp.float32),
        pltpu.VMEM((8, _KPAD), jnp.float32),
        pltpu.VMEM((_NV, _L), jnp.int32),
        pltpu.VMEM((_NV, _L), jnp.float32),
        pltpu.SMEM((1,), jnp.int32),
    ],
    compiler_params=pltpu.CompilerParams(needs_layout_passes=False),
)
def _nms_sc(tab_hbm, si_hbm, ss_hbm, out_hbm, tab_v, si_v, ss_v, sup_v, kept_v,
            mp_v, mh_v, nk_s):
    cid = lax.axis_index("c")
    sid = lax.axis_index("s")

    @pl.when((cid == 0) & (sid == 0))
    def _():
        zero16 = jnp.zeros((_L,), jnp.float32)
        for r in range(8):
            def _z(j, _, r=r):
                kept_v[r, pl.ds(j * _L, _L)] = zero16
                return 0
            lax.fori_loop(0, _KPAD // _L, _z, 0)

        rid = lax.iota(jnp.int32, _L)
        rowc = [jnp.full((_L,), r, jnp.int32) for r in range(6)]
        bfly = [rid ^ d for d in (1, 2, 4, 8)]
        nk_s[0] = jnp.int32(0)
        pltpu.sync_copy(tab_hbm, tab_v)
        pltpu.sync_copy(si_hbm, si_v)
        pltpu.sync_copy(ss_hbm, ss_v)

        sbase = [rid * _CP + jnp.int32(v * _L * _CP) for v in range(_NV)]
        for v in range(_NV):
            mp_v[v, :] = sbase[v]
            mh_v[v, :] = plsc.load_gather(ss_v, [sbase[v]])

        def chunk(c, carry):
            @pl.when(nk_s[0] < _DETECTIONS)
            def _():
                nk0 = nk_s[0]
                posv = [mp_v[v, :] for v in range(_NV)]
                hsc = [mh_v[v, :] for v in range(_NV)]

                povec = jnp.zeros((_L,), jnp.int32)
                svec = jnp.full((_L,), -2.0, jnp.float32)
                for t in range(_L):
                    acc = list(hsc)
                    while len(acc) > 1:
                        acc = [jnp.maximum(acc[2 * a], acc[2 * a + 1])
                               for a in range(len(acc) // 2)]
                    gm = acc[0]
                    for bf in bfly:
                        gm = jnp.maximum(gm, _perm(gm, bf))
                    m = [hsc[v] == gm for v in range(_NV)]
                    cnt = [plsc.all_reduce_population_count(m[v])
                           for v in range(_NV)]
                    ffs = [plsc.all_reduce_ffs(m[v]) for v in range(_NV)]
                    sv = ffs[_NV - 1] + (_NV - 1) * _L
                    for v in reversed(range(_NV - 1)):
                        sv = jnp.where(cnt[v] > 0, ffs[v] + v * _L, sv)
                    psel = posv[_NV - 1]
                    for v in reversed(range(_NV - 1)):
                        psel = jnp.where(sv < (v + 1) * _L, posv[v], psel)
                    ppos = _perm(psel, sv & (_L - 1))
                    povec = jnp.where(rid == t, ppos, povec)
                    svec = jnp.where(rid == t, gm, svec)
                    for v in range(_NV):
                        pop = (rid + v * _L) == sv
                        posv[v] = posv[v] + pop.astype(jnp.int32)
                        hs_new = plsc.load_gather(ss_v, [posv[v]])
                        hsc[v] = jnp.where(pop, hs_new, hsc[v])

                for v in range(_NV):
                    mp_v[v, :] = posv[v]
                    mh_v[v, :] = hsc[v]

                ovec = plsc.load_gather(si_v, [povec])
                x1 = plsc.load_gather(tab_v, [rowc[0], ovec])
                y1 = plsc.load_gather(tab_v, [rowc[1], ovec])
                x2 = plsc.load_gather(tab_v, [rowc[2], ovec])
                y2 = plsc.load_gather(tab_v, [rowc[3], ovec])
                s = svec
                area = (x2 - x1) * (y2 - y1)

                sup = jnp.where(s <= jnp.float32(_SCORE_THRESH),
                                jnp.float32(1.0), jnp.float32(0.0))

                def kbody(k, sup):
                    for u in range(4):
                        kv = jnp.full((_L,), 4 * k + u, jnp.int32)
                        kx1 = plsc.load_gather(kept_v, [rowc[0], kv])
                        ky1 = plsc.load_gather(kept_v, [rowc[1], kv])
                        kx2 = plsc.load_gather(kept_v, [rowc[2], kv])
                        ky2 = plsc.load_gather(kept_v, [rowc[3], kv])
                        ka = plsc.load_gather(kept_v, [rowc[5], kv])
                        iou = _iou_vs_chunk(kx1, ky1, kx2, ky2, ka,
                                            x1, y1, x2, y2, area)
                        sup = jnp.where(iou > jnp.float32(_NMS_THRESH),
                                        jnp.float32(1.0), sup)
                    return sup

                sup = lax.fori_loop(0, (nk0 + 3) // 4, kbody, sup)
                sup_v[...] = sup

                nk = nk0
                for i in range(_L):
                    supc = sup_v[...]
                    keep = (supc[i] == jnp.float32(0.0)) & (nk < _DETECTIONS)

                    @pl.when(keep)
                    def _(i=i, nk=nk):
                        bx1 = x1[i]
                        by1 = y1[i]
                        bx2 = x2[i]
                        by2 = y2[i]
                        ba = area[i]
                        lane = rid == i
                        nkv = jnp.full((_L,), nk, jnp.int32)
                        plsc.store_scatter(kept_v, [rowc[0], nkv], x1, mask=lane)
                        plsc.store_scatter(kept_v, [rowc[1], nkv], y1, mask=lane)
                        plsc.store_scatter(kept_v, [rowc[2], nkv], x2, mask=lane)
                        plsc.store_scatter(kept_v, [rowc[3], nkv], y2, mask=lane)
                        plsc.store_scatter(kept_v, [rowc[4], nkv], s, mask=lane)
                        plsc.store_scatter(kept_v, [rowc[5], nkv], area, mask=lane)
                        iou = _iou_vs_chunk(bx1, by1, bx2, by2, ba,
                                            x1, y1, x2, y2, area)
                        sup_v[...] = jnp.where(iou > jnp.float32(_NMS_THRESH),
                                               jnp.float32(1.0), sup_v[...])

                    nk = jnp.where(keep, nk + jnp.int32(1), nk)

                nk_s[0] = nk

                @pl.when(svec[0] <= jnp.float32(_SCORE_THRESH))
                def _():
                    nk_s[0] = jnp.int32(_DETECTIONS)

            return carry

        lax.fori_loop(0, _NCHUNK, chunk, jnp.int32(0))
        pltpu.sync_copy(kept_v, out_hbm)


def kernel(boxes, scores):
    sp = jnp.pad(scores, (0, _NTOT - _N), constant_values=-1.0)
    iv = jnp.arange(_NTOT, dtype=jnp.int32).reshape(_R, _C)
    sk, si = lax.sort_key_val(-sp.reshape(_R, _C), iv)
    pad_s = jnp.full((_R, _CP - _C), -2.0, jnp.float32)
    pad_i = jnp.full((_R, _CP - _C), _NTOT - 1, jnp.int32)
    ss = jnp.concatenate([-sk, pad_s], axis=1).reshape(-1)
    sip = jnp.concatenate([si, pad_i], axis=1).reshape(-1)
    b5 = jnp.concatenate([boxes, scores[:, None]], axis=1)
    tab = jnp.concatenate([jnp.pad(b5[:, :4], ((0, _NTOT - _N), (0, 0))),
                           sp[:, None]], axis=1).T
    out = _nms_sc(tab, sip, ss)
    return out[:5, :_DETECTIONS].T


# speedup vs baseline: 1.2393x; 1.1006x over previous
import functools

import jax
import jax.numpy as jnp
from jax import lax
from jax.experimental import pallas as pl
from jax.experimental.pallas import tpu as pltpu
from jax.experimental.pallas import tpu_sc as plsc

_SCORE_THRESH = 0.25
_NMS_THRESH = 0.45
_DETECTIONS = 300
_N = 5000
_L = 16
_NPAD = 5008
_NCHUNK = _NPAD // _L
_KPAD = 304

_mesh = plsc.VectorSubcoreMesh(core_axis_name="c", subcore_axis_name="s")


def _iou_vs_chunk(bx1, by1, bx2, by2, barea, x1, y1, x2, y2, area):
    ltx = jnp.maximum(bx1, x1)
    lty = jnp.maximum(by1, y1)
    rbx = jnp.minimum(bx2, x2)
    rby = jnp.minimum(by2, y2)
    w = jnp.maximum(rbx - ltx, jnp.float32(0.0))
    h = jnp.maximum(rby - lty, jnp.float32(0.0))
    inter = w * h
    return inter / (barea + area - inter + jnp.float32(1e-7))


@functools.partial(
    pl.kernel,
    out_type=jax.ShapeDtypeStruct((8, _KPAD), jnp.float32),
    mesh=_mesh,
    scratch_types=[
        pltpu.VMEM((5, _NPAD), jnp.float32),
        pltpu.VMEM((_NPAD,), jnp.int32),
        pltpu.VMEM((_L,), jnp.float32),
        pltpu.VMEM((8, _KPAD), jnp.float32),
        pltpu.SMEM((1,), jnp.int32),
    ],
    compiler_params=pltpu.CompilerParams(needs_layout_passes=False),
)
def _nms_sc(b_hbm, ord_hbm, out_hbm, tab_v, ord_v, sup_v, kept_v, nk_s):
    cid = lax.axis_index("c")
    sid = lax.axis_index("s")

    @pl.when((cid == 0) & (sid == 0))
    def _():
        zero16 = jnp.zeros((_L,), jnp.float32)
        for r in range(8):
            def _z(j, _, r=r):
                kept_v[r, pl.ds(j * _L, _L)] = zero16
                return 0
            lax.fori_loop(0, _KPAD // _L, _z, 0)

        rid = lax.iota(jnp.int32, _L)
        rowc = [jnp.full((_L,), r, jnp.int32) for r in range(6)]
        nk_s[0] = jnp.int32(0)
        pltpu.sync_copy(b_hbm, tab_v)
        pltpu.sync_copy(ord_hbm, ord_v)

        def chunk(c, carry):
            @pl.when(nk_s[0] < _DETECTIONS)
            def _():
                nk0 = nk_s[0]
                base = c * _L
                ovec = ord_v[pl.ds(base, _L)]

                x1 = plsc.load_gather(tab_v, [rowc[0], ovec])
                y1 = plsc.load_gather(tab_v, [rowc[1], ovec])
                x2 = plsc.load_gather(tab_v, [rowc[2], ovec])
                y2 = plsc.load_gather(tab_v, [rowc[3], ovec])
                s = plsc.load_gather(tab_v, [rowc[4], ovec])
                area = (x2 - x1) * (y2 - y1)

                sup = jnp.where(s <= jnp.float32(_SCORE_THRESH),
                                jnp.float32(1.0), jnp.float32(0.0))

                def kbody(k, sup):
                    kv = jnp.full((_L,), k, jnp.int32)
                    kx1 = plsc.load_gather(kept_v, [rowc[0], kv])
                    ky1 = plsc.load_gather(kept_v, [rowc[1], kv])
                    kx2 = plsc.load_gather(kept_v, [rowc[2], kv])
                    ky2 = plsc.load_gather(kept_v, [rowc[3], kv])
                    ka = (kx2 - kx1) * (ky2 - ky1)
                    iou = _iou_vs_chunk(kx1, ky1, kx2, ky2, ka,
                                        x1, y1, x2, y2, area)
                    return jnp.where(iou > jnp.float32(_NMS_THRESH),
                                     jnp.float32(1.0), sup)

                sup = lax.fori_loop(0, nk0, kbody, sup)
                sup_v[...] = sup

                nk = nk0
                for i in range(_L):
                    supc = sup_v[...]
                    keep = (supc[i] == jnp.float32(0.0)) & (nk < _DETECTIONS)

                    @pl.when(keep)
                    def _(i=i, nk=nk):
                        bx1 = x1[i]
                        by1 = y1[i]
                        bx2 = x2[i]
                        by2 = y2[i]
                        ba = area[i]
                        lane = rid == i
                        nkv = jnp.full((_L,), nk, jnp.int32)
                        plsc.store_scatter(kept_v, [rowc[0], nkv], x1, mask=lane)
                        plsc.store_scatter(kept_v, [rowc[1], nkv], y1, mask=lane)
                        plsc.store_scatter(kept_v, [rowc[2], nkv], x2, mask=lane)
                        plsc.store_scatter(kept_v, [rowc[3], nkv], y2, mask=lane)
                        plsc.store_scatter(kept_v, [rowc[4], nkv], s, mask=lane)
                        plsc.store_scatter(kept_v, [rowc[5], nkv], area, mask=lane)
                        iou = _iou_vs_chunk(bx1, by1, bx2, by2, ba,
                                            x1, y1, x2, y2, area)
                        sup_v[...] = jnp.where(iou > jnp.float32(_NMS_THRESH),
                                               jnp.float32(1.0), sup_v[...])

                    nk = jnp.where(keep, nk + jnp.int32(1), nk)

                nk_s[0] = nk

                @pl.when(s[0] <= jnp.float32(_SCORE_THRESH))
                def _():
                    nk_s[0] = jnp.int32(_DETECTIONS)

            return carry

        lax.fori_loop(0, _NCHUNK, chunk, jnp.int32(0))
        pltpu.sync_copy(kept_v, out_hbm)


def kernel(boxes, scores):
    order = jnp.argsort(-scores).astype(jnp.int32)
    b5 = jnp.concatenate([boxes, scores[:, None]], axis=1)
    b5 = jnp.pad(b5, ((0, _NPAD - _N), (0, 0))).T
    order_p = jnp.pad(order, (0, _NPAD - _N), constant_values=_N)
    out = _nms_sc(b5, order_p)
    return out[:5, :_DETECTIONS].T
